# Initial kernel scaffold; baseline (speedup 1.0000x reference)
import jax
import jax.numpy as jnp
from jax.experimental import pallas as pl


def kernel(feats, edge_index, W1, b1, W2, b2, Wout, bout):
    raise NotImplementedError("write your pallas kernel here")


# speedup vs baseline: 2.1929x; 2.1929x over previous
import functools

import jax
import jax.numpy as jnp
from jax import lax
from jax.experimental import pallas as pl
from jax.experimental.pallas import tpu as pltpu
from jax.experimental.pallas import tpu_sc as plsc

N = 10000
D = 128
E = 320000

NP = 10240
NW = 32
CHUNK = 128
EP = 327680
EW = EP // NW
NCHUNK = EW // CHUNK
RPT = NP // 16

_MESH = plsc.VectorSubcoreMesh(core_axis_name="c", subcore_axis_name="s")
_SC_PARAMS = pltpu.CompilerParams(needs_layout_passes=False)


@functools.partial(
    pl.kernel,
    out_type=jax.ShapeDtypeStruct((NW, 2, NP), jnp.float32),
    mesh=_MESH,
    compiler_params=_SC_PARAMS,
    scratch_types=[
        pltpu.VMEM((CHUNK,), jnp.int32),
        pltpu.VMEM((CHUNK,), jnp.int32),
        pltpu.VMEM((NP,), jnp.float32),
        pltpu.VMEM((NP,), jnp.float32),
    ],
)
def _deg_kernel(srcp, dstp, out, sidx, didx, oacc, iacc):
    c = lax.axis_index("c")
    s = lax.axis_index("s")
    wid = s * 2 + c

    zeros = jnp.zeros((16,), jnp.float32)

    def _zero(i, carry):
        oacc[pl.ds(i * 16, 16)] = zeros
        iacc[pl.ds(i * 16, 16)] = zeros
        return carry

    lax.fori_loop(0, NP // 16, _zero, 0)

    ones = jnp.ones((16,), jnp.float32)

    def _chunk(g, carry):
        base = wid * EW + g * CHUNK
        pltpu.sync_copy(srcp.at[pl.ds(base, CHUNK)], sidx)
        pltpu.sync_copy(dstp.at[pl.ds(base, CHUNK)], didx)
        for j in range(CHUNK // 16):
            plsc.addupdate_scatter(oacc, [sidx[pl.ds(j * 16, 16)]], ones)
            plsc.addupdate_scatter(iacc, [didx[pl.ds(j * 16, 16)]], ones)
        return carry

    lax.fori_loop(0, NCHUNK, _chunk, 0)

    pltpu.sync_copy(oacc, out.at[wid, 0])
    pltpu.sync_copy(iacc, out.at[wid, 1])


@functools.partial(
    pl.kernel,
    out_type=jax.ShapeDtypeStruct((2, NP, D), jnp.float32),
    mesh=_MESH,
    compiler_params=_SC_PARAMS,
    scratch_types=[
        pltpu.VMEM((CHUNK,), jnp.int32),
        pltpu.VMEM((CHUNK,), jnp.int32),
        pltpu.VMEM((CHUNK, D), jnp.float32),
        pltpu.VMEM_SHARED((NP, D), jnp.float32),
        pltpu.SemaphoreType.DMA,
    ],
)
def _agg_kernel(x, srcp, dstp, out, sidx, didx, rows, acc, sem):
    c = lax.axis_index("c")
    s = lax.axis_index("s")
    wid = s * 2 + c

    zeros = jnp.zeros((16,), jnp.float32)

    def _zrow(i, carry):
        for j in range(D // 16):
            rows[i, pl.ds(j * 16, 16)] = zeros
        return carry

    lax.fori_loop(0, CHUNK, _zrow, 0)
    for r in range(RPT // CHUNK):
        pltpu.sync_copy(rows, acc.at[pl.ds(s * RPT + r * CHUNK, CHUNK)])
    plsc.subcore_barrier()

    def _chunk(g, carry):
        base = wid * EW + g * CHUNK
        pltpu.sync_copy(srcp.at[pl.ds(base, CHUNK)], sidx)
        pltpu.sync_copy(dstp.at[pl.ds(base, CHUNK)], didx)
        pltpu.async_copy(x.at[sidx], rows, sem).wait()
        pltpu.sync_copy(rows, acc.at[didx], add=True)
        return carry

    lax.fori_loop(0, NCHUNK, _chunk, 0)
    plsc.subcore_barrier()

    for r in range(RPT // CHUNK):
        off = s * RPT + r * CHUNK
        pltpu.sync_copy(acc.at[pl.ds(off, CHUNK)], out.at[c, pl.ds(off, CHUNK)])


RB = 1024
GRID = NP // RB


def _rowmask(i):
    rid = i * RB + lax.broadcasted_iota(jnp.int32, (RB, 1), 0)
    return (rid < N).astype(jnp.float32)


def _norms_mm1_body(deg_ref, f_ref, w_ref, norms_ref, y_ref):
    i = pl.program_id(0)
    deg = jnp.sum(deg_ref[...], axis=0)
    norm = lax.rsqrt(jnp.maximum(deg, 1.0))
    norms_ref[...] = norm
    y = jnp.dot(f_ref[...], w_ref[...], preferred_element_type=jnp.float32)
    y_ref[...] = y * norm[0][:, None] * _rowmask(i)


def _norms_mm1(deg_parts, featsp, W1):
    return pl.pallas_call(
        _norms_mm1_body,
        grid=(GRID,),
        in_specs=[
            pl.BlockSpec((NW, 2, RB), lambda i: (0, 0, i)),
            pl.BlockSpec((RB, D), lambda i: (i, 0)),
            pl.BlockSpec((D, D), lambda i: (0, 0)),
        ],
        out_specs=[
            pl.BlockSpec((2, RB), lambda i: (0, i)),
            pl.BlockSpec((RB, D), lambda i: (i, 0)),
        ],
        out_shape=[
            jax.ShapeDtypeStruct((2, NP), jnp.float32),
            jax.ShapeDtypeStruct((NP, D), jnp.float32),
        ],
    )(deg_parts, featsp, W1)


def _layer_mm2_body(p_ref, n_ref, b_ref, w_ref, h_ref, y_ref):
    i = pl.program_id(0)
    agg = p_ref[0] + p_ref[1]
    nrm = n_ref[...]
    h = jnp.maximum(agg * nrm[1][:, None] + b_ref[...][None, :], 0.0)
    h = h * _rowmask(i)
    h_ref[...] = h
    y = jnp.dot(h, w_ref[...], preferred_element_type=jnp.float32)
    y_ref[...] = y * nrm[0][:, None]


def _layer_mm2(agg_parts, norms, b1, W2):
    return pl.pallas_call(
        _layer_mm2_body,
        grid=(GRID,),
        in_specs=[
            pl.BlockSpec((2, RB, D), lambda i: (0, i, 0)),
            pl.BlockSpec((2, RB), lambda i: (0, i)),
            pl.BlockSpec((D,), lambda i: (0,)),
            pl.BlockSpec((D, D), lambda i: (0, 0)),
        ],
        out_specs=[
            pl.BlockSpec((RB, D), lambda i: (i, 0)),
            pl.BlockSpec((RB, D), lambda i: (i, 0)),
        ],
        out_shape=[
            jax.ShapeDtypeStruct((NP, D), jnp.float32),
            jax.ShapeDtypeStruct((NP, D), jnp.float32),
        ],
    )(agg_parts, norms, b1, W2)


def _layer2_body(p_ref, n_ref, b_ref, h_ref):
    i = pl.program_id(0)
    agg = p_ref[0] + p_ref[1]
    nrm = n_ref[...]
    h = jnp.maximum(agg * nrm[1][:, None] + b_ref[...][None, :], 0.0)
    h_ref[...] = h * _rowmask(i)


def _layer2(agg_parts, norms, b2):
    return pl.pallas_call(
        _layer2_body,
        grid=(GRID,),
        in_specs=[
            pl.BlockSpec((2, RB, D), lambda i: (0, i, 0)),
            pl.BlockSpec((2, RB), lambda i: (0, i)),
            pl.BlockSpec((D,), lambda i: (0,)),
        ],
        out_specs=pl.BlockSpec((RB, D), lambda i: (i, 0)),
        out_shape=jax.ShapeDtypeStruct((NP, D), jnp.float32),
    )(agg_parts, norms, b2)


def _final_body(p1_ref, p2_ref, w_ref, b_ref, o_ref):
    a1 = p1_ref[0] + p1_ref[1]
    a2 = p2_ref[0] + p2_ref[1]
    w = w_ref[...]
    o = jnp.dot(a1, w[:D], preferred_element_type=jnp.float32)
    o += jnp.dot(a2, w[D:], preferred_element_type=jnp.float32)
    o_ref[...] = o + b_ref[...][None, :]


def _final(j1, j2, Wout, bout):
    return pl.pallas_call(
        _final_body,
        grid=(GRID,),
        in_specs=[
            pl.BlockSpec((2, RB, D), lambda i: (0, i, 0)),
            pl.BlockSpec((2, RB, D), lambda i: (0, i, 0)),
            pl.BlockSpec((2 * D, D), lambda i: (0, 0)),
            pl.BlockSpec((D,), lambda i: (0,)),
        ],
        out_specs=pl.BlockSpec((RB, D), lambda i: (i, 0)),
        out_shape=jax.ShapeDtypeStruct((N, D), jnp.float32),
    )(j1, j2, Wout, bout)


def kernel(feats, edge_index, W1, b1, W2, b2, Wout, bout):
    featsp = jnp.pad(feats, ((0, NP - N), (0, 0)))
    srcp = jnp.pad(edge_index[0], (0, EP - E), constant_values=N)
    dstp = jnp.pad(edge_index[1], (0, EP - E), constant_values=N)

    deg_parts = _deg_kernel(srcp, dstp)
    norms, y1 = _norms_mm1(deg_parts, featsp, W1)
    agg1 = _agg_kernel(y1, srcp, dstp)
    h1, y2 = _layer_mm2(agg1, norms, b1, W2)
    agg2 = _agg_kernel(y2, srcp, dstp)
    h2 = _layer2(agg2, norms, b2)
    j1 = _agg_kernel(h1, srcp, dstp)
    j2 = _agg_kernel(h2, srcp, dstp)
    return _final(j1, j2, Wout, bout)


# speedup vs baseline: 2.8097x; 1.2813x over previous
import functools

import jax
import jax.numpy as jnp
from jax import lax
from jax.experimental import pallas as pl
from jax.experimental.pallas import tpu as pltpu
from jax.experimental.pallas import tpu_sc as plsc

N = 10000
D = 128
E = 320000

NP = 10240
NW = 32
CHUNK = 128
EP = 327680
EW = EP // NW
NCHUNK = EW // CHUNK
RPT = NP // 16

_MESH = plsc.VectorSubcoreMesh(core_axis_name="c", subcore_axis_name="s")
_SC_PARAMS = pltpu.CompilerParams(needs_layout_passes=False)


@functools.partial(
    pl.kernel,
    out_type=jax.ShapeDtypeStruct((NW, 2, NP), jnp.float32),
    mesh=_MESH,
    compiler_params=_SC_PARAMS,
    scratch_types=[
        pltpu.VMEM((NCHUNK, CHUNK), jnp.int32),
        pltpu.VMEM((NCHUNK, CHUNK), jnp.int32),
        pltpu.VMEM((NP,), jnp.float32),
        pltpu.VMEM((NP,), jnp.float32),
    ],
)
def _deg_kernel(srcp, dstp, out, sidx, didx, oacc, iacc):
    c = lax.axis_index("c")
    s = lax.axis_index("s")
    wid = s * 2 + c

    zeros = jnp.zeros((16,), jnp.float32)

    def _zero(i, carry):
        oacc[pl.ds(i * 16, 16)] = zeros
        iacc[pl.ds(i * 16, 16)] = zeros
        return carry

    lax.fori_loop(0, NP // 16, _zero, 0)

    pltpu.sync_copy(srcp.at[wid], sidx)
    pltpu.sync_copy(dstp.at[wid], didx)

    ones = jnp.ones((16,), jnp.float32)

    def _chunk(g, carry):
        for j in range(CHUNK // 16):
            plsc.addupdate_scatter(oacc, [sidx[g, pl.ds(j * 16, 16)]], ones)
            plsc.addupdate_scatter(iacc, [didx[g, pl.ds(j * 16, 16)]], ones)
        return carry

    lax.fori_loop(0, NCHUNK, _chunk, 0)

    pltpu.sync_copy(oacc, out.at[wid, 0])
    pltpu.sync_copy(iacc, out.at[wid, 1])


NBUF = 2


@functools.partial(
    pl.kernel,
    out_type=jax.ShapeDtypeStruct((2, NP, D), jnp.float32),
    mesh=_MESH,
    compiler_params=_SC_PARAMS,
    scratch_types=[
        pltpu.VMEM((NCHUNK, CHUNK), jnp.int32),
        pltpu.VMEM((NBUF, CHUNK), jnp.int32),
        pltpu.VMEM((NBUF, CHUNK, D), jnp.float32),
        pltpu.VMEM_SHARED((NP, D), jnp.float32),
        pltpu.SemaphoreType.DMA((NBUF,)),
        pltpu.SemaphoreType.DMA((NBUF,)),
    ],
)
def _agg_kernel(x, srcp, dstp, out, sidx, didx, rows, acc, gsem, dsem):
    c = lax.axis_index("c")
    s = lax.axis_index("s")
    wid = s * 2 + c

    zeros = jnp.zeros((16,), jnp.float32)

    def _zrow(i, carry):
        for j in range(D // 16):
            rows[0, i, pl.ds(j * 16, 16)] = zeros
        return carry

    lax.fori_loop(0, CHUNK, _zrow, 0)
    for r in range(RPT // CHUNK):
        pltpu.sync_copy(rows.at[0], acc.at[pl.ds(s * RPT + r * CHUNK, CHUNK)])
    pltpu.sync_copy(srcp.at[wid], sidx)
    plsc.subcore_barrier()

    for b in range(NBUF):
        pltpu.async_copy(dstp.at[wid, b], didx.at[b], dsem.at[b])
        pltpu.async_copy(x.at[sidx.at[b]], rows.at[b], gsem.at[b])

    def _group(gg, carry):
        g0 = gg * NBUF
        for b in range(NBUF):
            g = g0 + b
            pltpu.make_async_copy(dstp.at[wid, 0], didx.at[b], dsem.at[b]).wait()
            pltpu.make_async_copy(x.at[sidx.at[0]], rows.at[b], gsem.at[b]).wait()
            pltpu.sync_copy(rows.at[b], acc.at[didx.at[b]], add=True)

            @pl.when(g + NBUF < NCHUNK)
            def _issue():
                pltpu.async_copy(dstp.at[wid, g + NBUF], didx.at[b], dsem.at[b])
                pltpu.async_copy(x.at[sidx.at[g + NBUF]], rows.at[b], gsem.at[b])

        return carry

    lax.fori_loop(0, NCHUNK // NBUF, _group, 0)
    plsc.subcore_barrier()

    for r in range(RPT // CHUNK):
        off = s * RPT + r * CHUNK
        pltpu.sync_copy(acc.at[pl.ds(off, CHUNK)], out.at[c, pl.ds(off, CHUNK)])


RB = 1024
GRID = NP // RB


def _rowmask(i):
    rid = i * RB + lax.broadcasted_iota(jnp.int32, (RB, 1), 0)
    return (rid < N).astype(jnp.float32)


def _norms_mm1_body(deg_ref, f_ref, w_ref, norms_ref, y_ref):
    i = pl.program_id(0)
    deg = jnp.sum(deg_ref[...], axis=0)
    norm = lax.rsqrt(jnp.maximum(deg, 1.0))
    norms_ref[...] = norm
    y = jnp.dot(f_ref[...], w_ref[...], preferred_element_type=jnp.float32)
    y_ref[...] = y * norm[0][:, None] * _rowmask(i)


def _norms_mm1(deg_parts, featsp, W1):
    return pl.pallas_call(
        _norms_mm1_body,
        grid=(GRID,),
        in_specs=[
            pl.BlockSpec((NW, 2, RB), lambda i: (0, 0, i)),
            pl.BlockSpec((RB, D), lambda i: (i, 0)),
            pl.BlockSpec((D, D), lambda i: (0, 0)),
        ],
        out_specs=[
            pl.BlockSpec((2, RB), lambda i: (0, i)),
            pl.BlockSpec((RB, D), lambda i: (i, 0)),
        ],
        out_shape=[
            jax.ShapeDtypeStruct((2, NP), jnp.float32),
            jax.ShapeDtypeStruct((NP, D), jnp.float32),
        ],
    )(deg_parts, featsp, W1)


def _layer_mm2_body(p_ref, n_ref, b_ref, w_ref, h_ref, y_ref):
    i = pl.program_id(0)
    agg = p_ref[0] + p_ref[1]
    nrm = n_ref[...]
    h = jnp.maximum(agg * nrm[1][:, None] + b_ref[...][None, :], 0.0)
    h = h * _rowmask(i)
    h_ref[...] = h
    y = jnp.dot(h, w_ref[...], preferred_element_type=jnp.float32)
    y_ref[...] = y * nrm[0][:, None]


def _layer_mm2(agg_parts, norms, b1, W2):
    return pl.pallas_call(
        _layer_mm2_body,
        grid=(GRID,),
        in_specs=[
            pl.BlockSpec((2, RB, D), lambda i: (0, i, 0)),
            pl.BlockSpec((2, RB), lambda i: (0, i)),
            pl.BlockSpec((D,), lambda i: (0,)),
            pl.BlockSpec((D, D), lambda i: (0, 0)),
        ],
        out_specs=[
            pl.BlockSpec((RB, D), lambda i: (i, 0)),
            pl.BlockSpec((RB, D), lambda i: (i, 0)),
        ],
        out_shape=[
            jax.ShapeDtypeStruct((NP, D), jnp.float32),
            jax.ShapeDtypeStruct((NP, D), jnp.float32),
        ],
    )(agg_parts, norms, b1, W2)


def _layer2_body(p_ref, n_ref, b_ref, h_ref):
    i = pl.program_id(0)
    agg = p_ref[0] + p_ref[1]
    nrm = n_ref[...]
    h = jnp.maximum(agg * nrm[1][:, None] + b_ref[...][None, :], 0.0)
    h_ref[...] = h * _rowmask(i)


def _layer2(agg_parts, norms, b2):
    return pl.pallas_call(
        _layer2_body,
        grid=(GRID,),
        in_specs=[
            pl.BlockSpec((2, RB, D), lambda i: (0, i, 0)),
            pl.BlockSpec((2, RB), lambda i: (0, i)),
            pl.BlockSpec((D,), lambda i: (0,)),
        ],
        out_specs=pl.BlockSpec((RB, D), lambda i: (i, 0)),
        out_shape=jax.ShapeDtypeStruct((NP, D), jnp.float32),
    )(agg_parts, norms, b2)


def _final_body(p1_ref, p2_ref, w_ref, b_ref, o_ref):
    a1 = p1_ref[0] + p1_ref[1]
    a2 = p2_ref[0] + p2_ref[1]
    w = w_ref[...]
    o = jnp.dot(a1, w[:D], preferred_element_type=jnp.float32)
    o += jnp.dot(a2, w[D:], preferred_element_type=jnp.float32)
    o_ref[...] = o + b_ref[...][None, :]


def _final(j1, j2, Wout, bout):
    return pl.pallas_call(
        _final_body,
        grid=(GRID,),
        in_specs=[
            pl.BlockSpec((2, RB, D), lambda i: (0, i, 0)),
            pl.BlockSpec((2, RB, D), lambda i: (0, i, 0)),
            pl.BlockSpec((2 * D, D), lambda i: (0, 0)),
            pl.BlockSpec((D,), lambda i: (0,)),
        ],
        out_specs=pl.BlockSpec((RB, D), lambda i: (i, 0)),
        out_shape=jax.ShapeDtypeStruct((N, D), jnp.float32),
    )(j1, j2, Wout, bout)


def kernel(feats, edge_index, W1, b1, W2, b2, Wout, bout):
    featsp = jnp.pad(feats, ((0, NP - N), (0, 0)))
    srcp = jnp.pad(edge_index[0], (0, EP - E), constant_values=N)
    dstp = jnp.pad(edge_index[1], (0, EP - E), constant_values=N)
    srcp = srcp.reshape(NW, NCHUNK, CHUNK)
    dstp = dstp.reshape(NW, NCHUNK, CHUNK)

    deg_parts = _deg_kernel(srcp, dstp)
    norms, y1 = _norms_mm1(deg_parts, featsp, W1)
    agg1 = _agg_kernel(y1, srcp, dstp)
    h1, y2 = _layer_mm2(agg1, norms, b1, W2)
    agg2 = _agg_kernel(y2, srcp, dstp)
    h2 = _layer2(agg2, norms, b2)
    j1 = _agg_kernel(h1, srcp, dstp)
    j2 = _agg_kernel(h2, srcp, dstp)
    return _final(j1, j2, Wout, bout)


# speedup vs baseline: 2.8137x; 1.0014x over previous
import functools

import jax
import jax.numpy as jnp
from jax import lax
from jax.experimental import pallas as pl
from jax.experimental.pallas import tpu as pltpu
from jax.experimental.pallas import tpu_sc as plsc

N = 10000
D = 128
E = 320000

NP = 10240
NW = 32
CHUNK = 128
EP = 327680
EW = EP // NW
NCHUNK = EW // CHUNK
RPT = NP // 16

_MESH = plsc.VectorSubcoreMesh(core_axis_name="c", subcore_axis_name="s")
_SC_PARAMS = pltpu.CompilerParams(needs_layout_passes=False)


@functools.partial(
    pl.kernel,
    out_type=jax.ShapeDtypeStruct((NW, 2, NP), jnp.float32),
    mesh=_MESH,
    compiler_params=_SC_PARAMS,
    scratch_types=[
        pltpu.VMEM((NCHUNK, CHUNK), jnp.int32),
        pltpu.VMEM((NCHUNK, CHUNK), jnp.int32),
        pltpu.VMEM((NP,), jnp.float32),
        pltpu.VMEM((NP,), jnp.float32),
    ],
)
def _deg_kernel(srcp, dstp, out, sidx, didx, oacc, iacc):
    c = lax.axis_index("c")
    s = lax.axis_index("s")
    wid = c * 16 + s

    zeros = jnp.zeros((16,), jnp.float32)

    def _zero(i, carry):
        oacc[pl.ds(i * 16, 16)] = zeros
        iacc[pl.ds(i * 16, 16)] = zeros
        return carry

    lax.fori_loop(0, NP // 16, _zero, 0)

    pltpu.sync_copy(srcp.at[wid], sidx)
    pltpu.sync_copy(dstp.at[wid], didx)

    ones = jnp.ones((16,), jnp.float32)

    def _chunk(g, carry):
        for j in range(CHUNK // 16):
            plsc.addupdate_scatter(oacc, [sidx[g, pl.ds(j * 16, 16)]], ones)
            plsc.addupdate_scatter(iacc, [didx[g, pl.ds(j * 16, 16)]], ones)
        return carry

    lax.fori_loop(0, NCHUNK, _chunk, 0)

    pltpu.sync_copy(oacc, out.at[wid, 0])
    pltpu.sync_copy(iacc, out.at[wid, 1])


NBUF = 2


@functools.partial(
    pl.kernel,
    out_type=jax.ShapeDtypeStruct((2, NP, D), jnp.float32),
    mesh=_MESH,
    compiler_params=_SC_PARAMS,
    scratch_types=[
        pltpu.VMEM((NCHUNK, CHUNK), jnp.int32),
        pltpu.VMEM((NBUF, CHUNK), jnp.int32),
        pltpu.VMEM((NBUF, CHUNK, D), jnp.float32),
        pltpu.VMEM_SHARED((NP, D), jnp.float32),
        pltpu.SemaphoreType.DMA((NBUF,)),
        pltpu.SemaphoreType.DMA((NBUF,)),
    ],
)
def _agg_kernel(x, srcp, dstp, out, sidx, didx, rows, acc, gsem, dsem):
    c = lax.axis_index("c")
    s = lax.axis_index("s")
    wid = c * 16 + s

    zeros = jnp.zeros((16,), jnp.float32)

    def _zrow(i, carry):
        for j in range(D // 16):
            rows[0, i, pl.ds(j * 16, 16)] = zeros
        return carry

    lax.fori_loop(0, CHUNK, _zrow, 0)
    for r in range(RPT // CHUNK):
        pltpu.sync_copy(rows.at[0], acc.at[pl.ds(s * RPT + r * CHUNK, CHUNK)])
    pltpu.sync_copy(srcp.at[wid], sidx)
    plsc.subcore_barrier()

    for b in range(NBUF):
        pltpu.async_copy(dstp.at[wid, b], didx.at[b], dsem.at[b])
        pltpu.async_copy(x.at[sidx.at[b]], rows.at[b], gsem.at[b])

    def _group(gg, carry):
        g0 = gg * NBUF
        for b in range(NBUF):
            g = g0 + b
            pltpu.make_async_copy(dstp.at[wid, 0], didx.at[b], dsem.at[b]).wait()
            pltpu.make_async_copy(x.at[sidx.at[0]], rows.at[b], gsem.at[b]).wait()
            pltpu.sync_copy(rows.at[b], acc.at[didx.at[b]], add=True)

            @pl.when(g + NBUF < NCHUNK)
            def _issue():
                pltpu.async_copy(dstp.at[wid, g + NBUF], didx.at[b], dsem.at[b])
                pltpu.async_copy(x.at[sidx.at[g + NBUF]], rows.at[b], gsem.at[b])

        return carry

    lax.fori_loop(0, NCHUNK // NBUF, _group, 0)
    plsc.subcore_barrier()

    for r in range(RPT // CHUNK):
        off = s * RPT + r * CHUNK
        pltpu.sync_copy(acc.at[pl.ds(off, CHUNK)], out.at[c, pl.ds(off, CHUNK)])


RB = 1024
GRID = NP // RB


def _rowmask(i):
    rid = i * RB + lax.broadcasted_iota(jnp.int32, (RB, 1), 0)
    return (rid < N).astype(jnp.float32)


def _norms_mm1_body(deg_ref, f_ref, w_ref, norms_ref, y_ref):
    i = pl.program_id(0)
    deg = jnp.sum(deg_ref[...], axis=0)
    norm = lax.rsqrt(jnp.maximum(deg, 1.0))
    norms_ref[...] = norm
    y = jnp.dot(f_ref[...], w_ref[...], preferred_element_type=jnp.float32)
    y_ref[...] = y * norm[0][:, None] * _rowmask(i)


def _norms_mm1(deg_parts, featsp, W1):
    return pl.pallas_call(
        _norms_mm1_body,
        grid=(GRID,),
        in_specs=[
            pl.BlockSpec((NW, 2, RB), lambda i: (0, 0, i)),
            pl.BlockSpec((RB, D), lambda i: (i, 0)),
            pl.BlockSpec((D, D), lambda i: (0, 0)),
        ],
        out_specs=[
            pl.BlockSpec((2, RB), lambda i: (0, i)),
            pl.BlockSpec((RB, D), lambda i: (i, 0)),
        ],
        out_shape=[
            jax.ShapeDtypeStruct((2, NP), jnp.float32),
            jax.ShapeDtypeStruct((NP, D), jnp.float32),
        ],
    )(deg_parts, featsp, W1)


def _layer_mm2_body(p_ref, n_ref, b_ref, w_ref, h_ref, y_ref):
    i = pl.program_id(0)
    agg = p_ref[0] + p_ref[1]
    nrm = n_ref[...]
    h = jnp.maximum(agg * nrm[1][:, None] + b_ref[...][None, :], 0.0)
    h = h * _rowmask(i)
    h_ref[...] = h
    y = jnp.dot(h, w_ref[...], preferred_element_type=jnp.float32)
    y_ref[...] = y * nrm[0][:, None]


def _layer_mm2(agg_parts, norms, b1, W2):
    return pl.pallas_call(
        _layer_mm2_body,
        grid=(GRID,),
        in_specs=[
            pl.BlockSpec((2, RB, D), lambda i: (0, i, 0)),
            pl.BlockSpec((2, RB), lambda i: (0, i)),
            pl.BlockSpec((D,), lambda i: (0,)),
            pl.BlockSpec((D, D), lambda i: (0, 0)),
        ],
        out_specs=[
            pl.BlockSpec((RB, D), lambda i: (i, 0)),
            pl.BlockSpec((RB, D), lambda i: (i, 0)),
        ],
        out_shape=[
            jax.ShapeDtypeStruct((NP, D), jnp.float32),
            jax.ShapeDtypeStruct((NP, D), jnp.float32),
        ],
    )(agg_parts, norms, b1, W2)


def _layer2_body(p_ref, n_ref, b_ref, h_ref):
    i = pl.program_id(0)
    agg = p_ref[0] + p_ref[1]
    nrm = n_ref[...]
    h = jnp.maximum(agg * nrm[1][:, None] + b_ref[...][None, :], 0.0)
    h_ref[...] = h * _rowmask(i)


def _layer2(agg_parts, norms, b2):
    return pl.pallas_call(
        _layer2_body,
        grid=(GRID,),
        in_specs=[
            pl.BlockSpec((2, RB, D), lambda i: (0, i, 0)),
            pl.BlockSpec((2, RB), lambda i: (0, i)),
            pl.BlockSpec((D,), lambda i: (0,)),
        ],
        out_specs=pl.BlockSpec((RB, D), lambda i: (i, 0)),
        out_shape=jax.ShapeDtypeStruct((NP, D), jnp.float32),
    )(agg_parts, norms, b2)


def _final_body(p1_ref, p2_ref, w_ref, b_ref, o_ref):
    a1 = p1_ref[0] + p1_ref[1]
    a2 = p2_ref[0] + p2_ref[1]
    w = w_ref[...]
    o = jnp.dot(a1, w[:D], preferred_element_type=jnp.float32)
    o += jnp.dot(a2, w[D:], preferred_element_type=jnp.float32)
    o_ref[...] = o + b_ref[...][None, :]


def _final(j1, j2, Wout, bout):
    return pl.pallas_call(
        _final_body,
        grid=(GRID,),
        in_specs=[
            pl.BlockSpec((2, RB, D), lambda i: (0, i, 0)),
            pl.BlockSpec((2, RB, D), lambda i: (0, i, 0)),
            pl.BlockSpec((2 * D, D), lambda i: (0, 0)),
            pl.BlockSpec((D,), lambda i: (0,)),
        ],
        out_specs=pl.BlockSpec((RB, D), lambda i: (i, 0)),
        out_shape=jax.ShapeDtypeStruct((N, D), jnp.float32),
    )(j1, j2, Wout, bout)


def kernel(feats, edge_index, W1, b1, W2, b2, Wout, bout):
    featsp = jnp.pad(feats, ((0, NP - N), (0, 0)))
    srcp = jnp.pad(edge_index[0], (0, EP - E), constant_values=N)
    dstp = jnp.pad(edge_index[1], (0, EP - E), constant_values=N)
    srcp = srcp.reshape(NW, NCHUNK, CHUNK)
    dstp = dstp.reshape(NW, NCHUNK, CHUNK)

    deg_parts = _deg_kernel(srcp, dstp)
    norms, y1 = _norms_mm1(deg_parts, featsp, W1)
    agg1 = _agg_kernel(y1, srcp, dstp)
    h1, y2 = _layer_mm2(agg1, norms, b1, W2)
    agg2 = _agg_kernel(y2, srcp, dstp)
    h2 = _layer2(agg2, norms, b2)
    j1 = _agg_kernel(h1, srcp, dstp)
    j2 = _agg_kernel(h2, srcp, dstp)
    return _final(j1, j2, Wout, bout)


# speedup vs baseline: 3.3932x; 1.2059x over previous
import functools

import jax
import jax.numpy as jnp
from jax import lax
from jax.experimental import pallas as pl
from jax.experimental.pallas import tpu as pltpu
from jax.experimental.pallas import tpu_sc as plsc

N = 10000
D = 128
E = 320000

NP = 10240
NW = 32
CHUNK = 128
EP = 327680
EW = EP // NW
NCHUNK = EW // CHUNK
RPT = NP // 16

_MESH = plsc.VectorSubcoreMesh(core_axis_name="c", subcore_axis_name="s")
_SC_PARAMS = pltpu.CompilerParams(needs_layout_passes=False)


@functools.partial(
    pl.kernel,
    out_type=jax.ShapeDtypeStruct((NW, 2, NP), jnp.float32),
    mesh=_MESH,
    compiler_params=_SC_PARAMS,
    scratch_types=[
        pltpu.VMEM((NCHUNK, CHUNK), jnp.int32),
        pltpu.VMEM((NCHUNK, CHUNK), jnp.int32),
        pltpu.VMEM((NP,), jnp.float32),
        pltpu.VMEM((NP,), jnp.float32),
    ],
)
def _deg_kernel(srcp, dstp, out, sidx, didx, oacc, iacc):
    c = lax.axis_index("c")
    s = lax.axis_index("s")
    wid = c * 16 + s

    zeros = jnp.zeros((16,), jnp.float32)

    def _zero(i, carry):
        oacc[pl.ds(i * 16, 16)] = zeros
        iacc[pl.ds(i * 16, 16)] = zeros
        return carry

    lax.fori_loop(0, NP // 16, _zero, 0)

    pltpu.sync_copy(srcp.at[wid], sidx)
    pltpu.sync_copy(dstp.at[wid], didx)

    ones = jnp.ones((16,), jnp.float32)

    def _chunk(g, carry):
        for j in range(CHUNK // 16):
            plsc.addupdate_scatter(oacc, [sidx[g, pl.ds(j * 16, 16)]], ones)
            plsc.addupdate_scatter(iacc, [didx[g, pl.ds(j * 16, 16)]], ones)
        return carry

    lax.fori_loop(0, NCHUNK, _chunk, 0)

    pltpu.sync_copy(oacc, out.at[wid, 0])
    pltpu.sync_copy(iacc, out.at[wid, 1])


NBUF = 2
IBUF = 4
NC0 = 128
NC1 = 32
CPP = NC0 + NC1
NCHUNKS_TOT = EP // CHUNK


@functools.partial(
    pl.kernel,
    out_type=jax.ShapeDtypeStruct((2, NP, D), jnp.float32),
    mesh=_MESH,
    compiler_params=_SC_PARAMS,
    scratch_types=[
        pltpu.VMEM((IBUF, CHUNK), jnp.int32),
        pltpu.VMEM((IBUF, CHUNK), jnp.int32),
        pltpu.VMEM((NBUF, CHUNK, D), jnp.float32),
        pltpu.VMEM_SHARED((NP, D), jnp.float32),
        pltpu.SemaphoreType.DMA((IBUF,)),
        pltpu.SemaphoreType.DMA((IBUF,)),
        pltpu.SemaphoreType.DMA((NBUF,)),
    ],
)
def _agg_kernel(x, srcp, dstp, out, sidx, didx, rows, acc, ssem, dsem, gsem):
    c = lax.axis_index("c")
    s = lax.axis_index("s")
    cbase = s * CPP + c * NC0
    nc = jnp.where(c == 0, NC0, NC1)

    zeros = jnp.zeros((16,), jnp.float32)

    def _zrow(i, carry):
        for j in range(D // 16):
            rows[0, i, pl.ds(j * 16, 16)] = zeros
        return carry

    lax.fori_loop(0, CHUNK, _zrow, 0)
    for r in range(RPT // CHUNK):
        pltpu.sync_copy(rows.at[0], acc.at[pl.ds(s * RPT + r * CHUNK, CHUNK)])
    plsc.subcore_barrier()

    def _issue_idx(g, slot):
        pltpu.async_copy(srcp.at[cbase + g], sidx.at[slot], ssem.at[slot])
        pltpu.async_copy(dstp.at[cbase + g], didx.at[slot], dsem.at[slot])

    def _wait_idx(slot):
        pltpu.make_async_copy(srcp.at[0], sidx.at[slot], ssem.at[slot]).wait()
        pltpu.make_async_copy(dstp.at[0], didx.at[slot], dsem.at[slot]).wait()

    def _issue_gather(g, slot, b):
        pltpu.async_copy(x.at[sidx.at[slot]], rows.at[b], gsem.at[b])

    def _wait_gather(b):
        pltpu.make_async_copy(x.at[sidx.at[0]], rows.at[b], gsem.at[b]).wait()

    for i in range(IBUF):
        @pl.when(i < nc)
        def _p1():
            _issue_idx(i, i)
    for b in range(NBUF):
        @pl.when(b < nc)
        def _p2():
            _wait_idx(b)
            _issue_gather(b, b, b)

    def _step(g, carry):
        b = lax.rem(g, NBUF)
        i = lax.rem(g, IBUF)
        _wait_gather(b)
        pltpu.sync_copy(rows.at[b], acc.at[didx.at[i]], add=True)

        @pl.when(g + IBUF < nc)
        def _refill():
            _issue_idx(g + IBUF, i)

        @pl.when(g + NBUF < nc)
        def _next():
            i2 = lax.rem(g + NBUF, IBUF)
            _wait_idx(i2)
            _issue_gather(g + NBUF, i2, b)

        return carry

    lax.fori_loop(0, nc, _step, 0)
    plsc.subcore_barrier()

    for r in range(RPT // CHUNK):
        off = s * RPT + r * CHUNK
        pltpu.sync_copy(acc.at[pl.ds(off, CHUNK)], out.at[c, pl.ds(off, CHUNK)])


RB = 1024
GRID = NP // RB


def _rowmask(i):
    rid = i * RB + lax.broadcasted_iota(jnp.int32, (RB, 1), 0)
    return (rid < N).astype(jnp.float32)


def _norms_mm1_body(deg_ref, f_ref, w_ref, norms_ref, y_ref):
    i = pl.program_id(0)
    deg = jnp.sum(deg_ref[...], axis=0)
    norm = lax.rsqrt(jnp.maximum(deg, 1.0))
    norms_ref[...] = norm
    y = jnp.dot(f_ref[...], w_ref[...], preferred_element_type=jnp.float32)
    y_ref[...] = y * norm[0][:, None] * _rowmask(i)


def _norms_mm1(deg_parts, featsp, W1):
    return pl.pallas_call(
        _norms_mm1_body,
        grid=(GRID,),
        in_specs=[
            pl.BlockSpec((NW, 2, RB), lambda i: (0, 0, i)),
            pl.BlockSpec((RB, D), lambda i: (i, 0)),
            pl.BlockSpec((D, D), lambda i: (0, 0)),
        ],
        out_specs=[
            pl.BlockSpec((2, RB), lambda i: (0, i)),
            pl.BlockSpec((RB, D), lambda i: (i, 0)),
        ],
        out_shape=[
            jax.ShapeDtypeStruct((2, NP), jnp.float32),
            jax.ShapeDtypeStruct((NP, D), jnp.float32),
        ],
    )(deg_parts, featsp, W1)


def _layer_mm2_body(p_ref, n_ref, b_ref, w_ref, h_ref, y_ref):
    i = pl.program_id(0)
    agg = p_ref[0] + p_ref[1]
    nrm = n_ref[...]
    h = jnp.maximum(agg * nrm[1][:, None] + b_ref[...][None, :], 0.0)
    h = h * _rowmask(i)
    h_ref[...] = h
    y = jnp.dot(h, w_ref[...], preferred_element_type=jnp.float32)
    y_ref[...] = y * nrm[0][:, None]


def _layer_mm2(agg_parts, norms, b1, W2):
    return pl.pallas_call(
        _layer_mm2_body,
        grid=(GRID,),
        in_specs=[
            pl.BlockSpec((2, RB, D), lambda i: (0, i, 0)),
            pl.BlockSpec((2, RB), lambda i: (0, i)),
            pl.BlockSpec((D,), lambda i: (0,)),
            pl.BlockSpec((D, D), lambda i: (0, 0)),
        ],
        out_specs=[
            pl.BlockSpec((RB, D), lambda i: (i, 0)),
            pl.BlockSpec((RB, D), lambda i: (i, 0)),
        ],
        out_shape=[
            jax.ShapeDtypeStruct((NP, D), jnp.float32),
            jax.ShapeDtypeStruct((NP, D), jnp.float32),
        ],
    )(agg_parts, norms, b1, W2)


def _layer2_body(p_ref, n_ref, b_ref, h_ref):
    i = pl.program_id(0)
    agg = p_ref[0] + p_ref[1]
    nrm = n_ref[...]
    h = jnp.maximum(agg * nrm[1][:, None] + b_ref[...][None, :], 0.0)
    h_ref[...] = h * _rowmask(i)


def _layer2(agg_parts, norms, b2):
    return pl.pallas_call(
        _layer2_body,
        grid=(GRID,),
        in_specs=[
            pl.BlockSpec((2, RB, D), lambda i: (0, i, 0)),
            pl.BlockSpec((2, RB), lambda i: (0, i)),
            pl.BlockSpec((D,), lambda i: (0,)),
        ],
        out_specs=pl.BlockSpec((RB, D), lambda i: (i, 0)),
        out_shape=jax.ShapeDtypeStruct((NP, D), jnp.float32),
    )(agg_parts, norms, b2)


def _final_body(p1_ref, p2_ref, w_ref, b_ref, o_ref):
    a1 = p1_ref[0] + p1_ref[1]
    a2 = p2_ref[0] + p2_ref[1]
    w = w_ref[...]
    o = jnp.dot(a1, w[:D], preferred_element_type=jnp.float32)
    o += jnp.dot(a2, w[D:], preferred_element_type=jnp.float32)
    o_ref[...] = o + b_ref[...][None, :]


def _final(j1, j2, Wout, bout):
    return pl.pallas_call(
        _final_body,
        grid=(GRID,),
        in_specs=[
            pl.BlockSpec((2, RB, D), lambda i: (0, i, 0)),
            pl.BlockSpec((2, RB, D), lambda i: (0, i, 0)),
            pl.BlockSpec((2 * D, D), lambda i: (0, 0)),
            pl.BlockSpec((D,), lambda i: (0,)),
        ],
        out_specs=pl.BlockSpec((RB, D), lambda i: (i, 0)),
        out_shape=jax.ShapeDtypeStruct((N, D), jnp.float32),
    )(j1, j2, Wout, bout)


def kernel(feats, edge_index, W1, b1, W2, b2, Wout, bout):
    featsp = jnp.pad(feats, ((0, NP - N), (0, 0)))
    srcp = jnp.pad(edge_index[0], (0, EP - E), constant_values=N)
    dstp = jnp.pad(edge_index[1], (0, EP - E), constant_values=N)
    srcp3 = srcp.reshape(NW, NCHUNK, CHUNK)
    dstp3 = dstp.reshape(NW, NCHUNK, CHUNK)
    srcp2 = srcp.reshape(NCHUNKS_TOT, CHUNK)
    dstp2 = dstp.reshape(NCHUNKS_TOT, CHUNK)

    deg_parts = _deg_kernel(srcp3, dstp3)
    norms, y1 = _norms_mm1(deg_parts, featsp, W1)
    agg1 = _agg_kernel(y1, srcp2, dstp2)
    h1, y2 = _layer_mm2(agg1, norms, b1, W2)
    agg2 = _agg_kernel(y2, srcp2, dstp2)
    h2 = _layer2(agg2, norms, b2)
    j1 = _agg_kernel(h1, srcp2, dstp2)
    j2 = _agg_kernel(h2, srcp2, dstp2)
    return _final(j1, j2, Wout, bout)


# speedup vs baseline: 3.4773x; 1.0248x over previous
import functools

import jax
import jax.numpy as jnp
from jax import lax
from jax.experimental import pallas as pl
from jax.experimental.pallas import tpu as pltpu
from jax.experimental.pallas import tpu_sc as plsc

N = 10000
D = 128
E = 320000

NP = 10240
NW = 32
CHUNK = 128
RPT = NP // 16

NC0 = 120
NC1 = 40
CPP = NC0 + NC1
EP = 16 * CPP * CHUNK
NCHUNKS_TOT = EP // CHUNK
EW_DEG = EP // NW
PKROWS = 2640

_MESH = plsc.VectorSubcoreMesh(core_axis_name="c", subcore_axis_name="s")
_SC_PARAMS = pltpu.CompilerParams(needs_layout_passes=False)


@functools.partial(
    pl.kernel,
    out_type=jax.ShapeDtypeStruct((NW, 2, NP), jnp.float32),
    mesh=_MESH,
    compiler_params=_SC_PARAMS,
    scratch_types=[
        pltpu.VMEM((EW_DEG,), jnp.int32),
        pltpu.VMEM((EW_DEG,), jnp.int32),
        pltpu.VMEM((NP,), jnp.float32),
        pltpu.VMEM((NP,), jnp.float32),
    ],
)
def _deg_kernel(srcp, dstp, out, sidx, didx, oacc, iacc):
    c = lax.axis_index("c")
    s = lax.axis_index("s")
    wid = c * 16 + s

    zeros = jnp.zeros((16,), jnp.float32)

    def _zero(i, carry):
        oacc[pl.ds(i * 16, 16)] = zeros
        iacc[pl.ds(i * 16, 16)] = zeros
        return carry

    lax.fori_loop(0, NP // 16, _zero, 0)

    pltpu.sync_copy(srcp.at[wid], sidx)
    pltpu.sync_copy(dstp.at[wid], didx)

    ones = jnp.ones((16,), jnp.float32)

    def _grp(g, carry):
        plsc.addupdate_scatter(oacc, [sidx[pl.ds(g * 16, 16)]], ones)
        plsc.addupdate_scatter(iacc, [didx[pl.ds(g * 16, 16)]], ones)
        return carry

    lax.fori_loop(0, EW_DEG // 16, _grp, 0)

    pltpu.sync_copy(oacc, out.at[wid, 0])
    pltpu.sync_copy(iacc, out.at[wid, 1])


NBUF = 2
PKW = CHUNK // 2


@functools.partial(
    pl.kernel,
    out_type=jax.ShapeDtypeStruct((2, NP, D), jnp.float32),
    mesh=_MESH,
    compiler_params=_SC_PARAMS,
    scratch_types=[
        pltpu.VMEM((NC0, CHUNK), jnp.int32),
        pltpu.VMEM((NBUF, CHUNK), jnp.int32),
        pltpu.VMEM((CHUNK,), jnp.int32),
        pltpu.VMEM((NBUF, CHUNK, D), jnp.float32),
        pltpu.VMEM_SHARED((NP, D), jnp.float32),
        pltpu.SemaphoreType.DMA((NBUF,)),
    ],
)
def _agg_kernel(x, pk, out, pkv, sidx, didx, rows, acc, gsem):
    c = lax.axis_index("c")
    s = lax.axis_index("s")
    cbase = s * CPP + c * NC0
    nc = jnp.where(c == 0, NC0, NC1)

    zeros = jnp.zeros((16,), jnp.float32)

    def _zrow(i, carry):
        for j in range(D // 16):
            rows[0, i, pl.ds(j * 16, 16)] = zeros
        return carry

    lax.fori_loop(0, CHUNK, _zrow, 0)
    for r in range(RPT // CHUNK):
        pltpu.sync_copy(rows.at[0], acc.at[pl.ds(s * RPT + r * CHUNK, CHUNK)])
    pltpu.sync_copy(pk.at[pl.ds(cbase, NC0)], pkv)
    plsc.subcore_barrier()

    def _unpack(g, col0, dst_ref):
        for j in range(PKW // 16):
            v = pkv[g, pl.ds(col0 + j * 16, 16)]
            dst_ref[pl.ds(j * 16, 16)] = v & 0xFFFF
            dst_ref[pl.ds(PKW + j * 16, 16)] = lax.shift_right_logical(v, 16)

    def _issue_gather(g, b):
        _unpack(g, 0, sidx.at[b])
        pltpu.async_copy(x.at[sidx.at[b]], rows.at[b], gsem.at[b])

    def _wait_gather(b):
        pltpu.make_async_copy(x.at[sidx.at[0]], rows.at[b], gsem.at[b]).wait()

    for b in range(NBUF):
        _issue_gather(b, b)

    def _step(g, carry):
        b = lax.rem(g, NBUF)
        _wait_gather(b)
        _unpack(g, PKW, didx)
        pltpu.sync_copy(rows.at[b], acc.at[didx], add=True)

        @pl.when(g + NBUF < nc)
        def _next():
            _issue_gather(g + NBUF, b)

        return carry

    lax.fori_loop(0, nc, _step, 0)
    plsc.subcore_barrier()

    for r in range(RPT // CHUNK):
        off = s * RPT + r * CHUNK
        pltpu.sync_copy(acc.at[pl.ds(off, CHUNK)], out.at[c, pl.ds(off, CHUNK)])


RB = 1024
GRID = NP // RB


def _rowmask(i):
    rid = i * RB + lax.broadcasted_iota(jnp.int32, (RB, 1), 0)
    return (rid < N).astype(jnp.float32)


def _norms_mm1_body(deg_ref, f_ref, w_ref, norms_ref, y_ref):
    i = pl.program_id(0)
    deg = jnp.sum(deg_ref[...], axis=0)
    norm = lax.rsqrt(jnp.maximum(deg, 1.0))
    norms_ref[...] = norm
    y = jnp.dot(f_ref[...], w_ref[...], preferred_element_type=jnp.float32)
    y_ref[...] = y * norm[0][:, None] * _rowmask(i)


def _norms_mm1(deg_parts, featsp, W1):
    return pl.pallas_call(
        _norms_mm1_body,
        grid=(GRID,),
        in_specs=[
            pl.BlockSpec((NW, 2, RB), lambda i: (0, 0, i)),
            pl.BlockSpec((RB, D), lambda i: (i, 0)),
            pl.BlockSpec((D, D), lambda i: (0, 0)),
        ],
        out_specs=[
            pl.BlockSpec((2, RB), lambda i: (0, i)),
            pl.BlockSpec((RB, D), lambda i: (i, 0)),
        ],
        out_shape=[
            jax.ShapeDtypeStruct((2, NP), jnp.float32),
            jax.ShapeDtypeStruct((NP, D), jnp.float32),
        ],
    )(deg_parts, featsp, W1)


def _layer_mm2_body(p_ref, n_ref, b_ref, w_ref, h_ref, y_ref):
    i = pl.program_id(0)
    agg = p_ref[0] + p_ref[1]
    nrm = n_ref[...]
    h = jnp.maximum(agg * nrm[1][:, None] + b_ref[...][None, :], 0.0)
    h = h * _rowmask(i)
    h_ref[...] = h
    y = jnp.dot(h, w_ref[...], preferred_element_type=jnp.float32)
    y_ref[...] = y * nrm[0][:, None]


def _layer_mm2(agg_parts, norms, b1, W2):
    return pl.pallas_call(
        _layer_mm2_body,
        grid=(GRID,),
        in_specs=[
            pl.BlockSpec((2, RB, D), lambda i: (0, i, 0)),
            pl.BlockSpec((2, RB), lambda i: (0, i)),
            pl.BlockSpec((D,), lambda i: (0,)),
            pl.BlockSpec((D, D), lambda i: (0, 0)),
        ],
        out_specs=[
            pl.BlockSpec((RB, D), lambda i: (i, 0)),
            pl.BlockSpec((RB, D), lambda i: (i, 0)),
        ],
        out_shape=[
            jax.ShapeDtypeStruct((NP, D), jnp.float32),
            jax.ShapeDtypeStruct((NP, D), jnp.float32),
        ],
    )(agg_parts, norms, b1, W2)


def _layer2_body(p_ref, n_ref, b_ref, h_ref):
    i = pl.program_id(0)
    agg = p_ref[0] + p_ref[1]
    nrm = n_ref[...]
    h = jnp.maximum(agg * nrm[1][:, None] + b_ref[...][None, :], 0.0)
    h_ref[...] = h * _rowmask(i)


def _layer2(agg_parts, norms, b2):
    return pl.pallas_call(
        _layer2_body,
        grid=(GRID,),
        in_specs=[
            pl.BlockSpec((2, RB, D), lambda i: (0, i, 0)),
            pl.BlockSpec((2, RB), lambda i: (0, i)),
            pl.BlockSpec((D,), lambda i: (0,)),
        ],
        out_specs=pl.BlockSpec((RB, D), lambda i: (i, 0)),
        out_shape=jax.ShapeDtypeStruct((NP, D), jnp.float32),
    )(agg_parts, norms, b2)


def _final_body(p1_ref, p2_ref, w_ref, b_ref, o_ref):
    a1 = p1_ref[0] + p1_ref[1]
    a2 = p2_ref[0] + p2_ref[1]
    w = w_ref[...]
    o = jnp.dot(a1, w[:D], preferred_element_type=jnp.float32)
    o += jnp.dot(a2, w[D:], preferred_element_type=jnp.float32)
    o_ref[...] = o + b_ref[...][None, :]


def _final(j1, j2, Wout, bout):
    return pl.pallas_call(
        _final_body,
        grid=(GRID,),
        in_specs=[
            pl.BlockSpec((2, RB, D), lambda i: (0, i, 0)),
            pl.BlockSpec((2, RB, D), lambda i: (0, i, 0)),
            pl.BlockSpec((2 * D, D), lambda i: (0, 0)),
            pl.BlockSpec((D,), lambda i: (0,)),
        ],
        out_specs=pl.BlockSpec((RB, D), lambda i: (i, 0)),
        out_shape=jax.ShapeDtypeStruct((N, D), jnp.float32),
    )(j1, j2, Wout, bout)


def kernel(feats, edge_index, W1, b1, W2, b2, Wout, bout):
    featsp = jnp.pad(feats, ((0, NP - N), (0, 0)))
    srcp = jnp.pad(edge_index[0], (0, EP - E), constant_values=N)
    dstp = jnp.pad(edge_index[1], (0, EP - E), constant_values=N)
    srcp3 = srcp.reshape(NW, EW_DEG)
    dstp3 = dstp.reshape(NW, EW_DEG)

    def _pack(a):
        a3 = a.reshape(NCHUNKS_TOT, 2, PKW)
        return a3[:, 0, :] | (a3[:, 1, :] << 16)

    pk = jnp.concatenate([_pack(srcp), _pack(dstp)], axis=1)
    pk = jnp.pad(pk, ((0, PKROWS - NCHUNKS_TOT), (0, 0)))

    deg_parts = _deg_kernel(srcp3, dstp3)
    norms, y1 = _norms_mm1(deg_parts, featsp, W1)
    agg1 = _agg_kernel(y1, pk)
    h1, y2 = _layer_mm2(agg1, norms, b1, W2)
    agg2 = _agg_kernel(y2, pk)
    h2 = _layer2(agg2, norms, b2)
    j1 = _agg_kernel(h1, pk)
    j2 = _agg_kernel(h2, pk)
    return _final(j1, j2, Wout, bout)


# speedup vs baseline: 3.4777x; 1.0001x over previous
import functools

import jax
import jax.numpy as jnp
from jax import lax
from jax.experimental import pallas as pl
from jax.experimental.pallas import tpu as pltpu
from jax.experimental.pallas import tpu_sc as plsc

N = 10000
D = 128
E = 320000

NP = 10240
NW = 32
CHUNK = 128
RPT = NP // 16

NC0 = 120
NC1 = 40
CPP = NC0 + NC1
EP = 16 * CPP * CHUNK
NCHUNKS_TOT = EP // CHUNK
EW_DEG = EP // NW
PKROWS = 2640

_MESH = plsc.VectorSubcoreMesh(core_axis_name="c", subcore_axis_name="s")
_SC_PARAMS = pltpu.CompilerParams(needs_layout_passes=False)


@functools.partial(
    pl.kernel,
    out_type=jax.ShapeDtypeStruct((NW, 2, NP), jnp.float32),
    mesh=_MESH,
    compiler_params=_SC_PARAMS,
    scratch_types=[
        pltpu.VMEM((EW_DEG,), jnp.int32),
        pltpu.VMEM((EW_DEG,), jnp.int32),
        pltpu.VMEM((NP,), jnp.float32),
        pltpu.VMEM((NP,), jnp.float32),
    ],
)
def _deg_kernel(srcp, dstp, out, sidx, didx, oacc, iacc):
    c = lax.axis_index("c")
    s = lax.axis_index("s")
    wid = c * 16 + s

    zeros = jnp.zeros((16,), jnp.float32)

    def _zero(i, carry):
        oacc[pl.ds(i * 16, 16)] = zeros
        iacc[pl.ds(i * 16, 16)] = zeros
        return carry

    lax.fori_loop(0, NP // 16, _zero, 0)

    pltpu.sync_copy(srcp.at[wid], sidx)
    pltpu.sync_copy(dstp.at[wid], didx)

    ones = jnp.ones((16,), jnp.float32)

    def _grp(g, carry):
        plsc.addupdate_scatter(oacc, [sidx[pl.ds(g * 16, 16)]], ones)
        plsc.addupdate_scatter(iacc, [didx[pl.ds(g * 16, 16)]], ones)
        return carry

    lax.fori_loop(0, EW_DEG // 16, _grp, 0)

    pltpu.sync_copy(oacc, out.at[wid, 0])
    pltpu.sync_copy(iacc, out.at[wid, 1])


NBUF = 2
PKW = CHUNK // 2


@functools.partial(
    pl.kernel,
    out_type=jax.ShapeDtypeStruct((2, NP, D), jnp.float32),
    mesh=_MESH,
    compiler_params=_SC_PARAMS,
    scratch_types=[
        pltpu.VMEM((NC0, CHUNK), jnp.int32),
        pltpu.VMEM((NBUF, CHUNK), jnp.int32),
        pltpu.VMEM((CHUNK,), jnp.int32),
        pltpu.VMEM((NBUF, CHUNK, D), jnp.float32),
        pltpu.VMEM_SHARED((NP, D), jnp.float32),
        pltpu.SemaphoreType.DMA((NBUF,)),
    ],
)
def _agg_kernel(x, pk, out, pkv, sidx, didx, rows, acc, gsem):
    c = lax.axis_index("c")
    s = lax.axis_index("s")
    cbase = s * CPP + c * NC0
    nc = jnp.where(c == 0, NC0, NC1)

    zeros = jnp.zeros((16,), jnp.float32)

    def _zrow(i, carry):
        for j in range(D // 16):
            rows[0, i, pl.ds(j * 16, 16)] = zeros
        return carry

    lax.fori_loop(0, CHUNK, _zrow, 0)
    for r in range(RPT // CHUNK):
        pltpu.sync_copy(rows.at[0], acc.at[pl.ds(s * RPT + r * CHUNK, CHUNK)])
    pltpu.sync_copy(pk.at[pl.ds(cbase, NC0)], pkv)
    plsc.subcore_barrier()

    def _unpack(g, col0, dst_ref):
        for j in range(PKW // 16):
            v = pkv[g, pl.ds(col0 + j * 16, 16)]
            dst_ref[pl.ds(j * 16, 16)] = v & 0xFFFF
            dst_ref[pl.ds(PKW + j * 16, 16)] = lax.shift_right_logical(v, 16)

    def _issue_gather(g, b):
        _unpack(g, 0, sidx.at[b])
        pltpu.async_copy(x.at[sidx.at[b]], rows.at[b], gsem.at[b])

    def _wait_gather(b):
        pltpu.make_async_copy(x.at[sidx.at[0]], rows.at[b], gsem.at[b]).wait()

    for b in range(NBUF):
        _issue_gather(b, b)

    def _group(gg, carry):
        for b in range(NBUF):
            g = gg * NBUF + b
            _wait_gather(b)
            _unpack(g, PKW, didx)
            pltpu.sync_copy(rows.at[b], acc.at[didx], add=True)

            @pl.when(g + NBUF < nc)
            def _next():
                _issue_gather(g + NBUF, b)

        return carry

    lax.fori_loop(0, nc // NBUF, _group, 0)
    plsc.subcore_barrier()

    for r in range(RPT // CHUNK):
        off = s * RPT + r * CHUNK
        pltpu.sync_copy(acc.at[pl.ds(off, CHUNK)], out.at[c, pl.ds(off, CHUNK)])


RB = 1024
GRID = NP // RB


def _rowmask(i):
    rid = i * RB + lax.broadcasted_iota(jnp.int32, (RB, 1), 0)
    return (rid < N).astype(jnp.float32)


def _norms_mm1_body(deg_ref, f_ref, w_ref, norms_ref, y_ref):
    i = pl.program_id(0)
    deg = jnp.sum(deg_ref[...], axis=0)
    norm = lax.rsqrt(jnp.maximum(deg, 1.0))
    norms_ref[...] = norm
    y = jnp.dot(f_ref[...], w_ref[...], preferred_element_type=jnp.float32)
    y_ref[...] = y * norm[0][:, None] * _rowmask(i)


def _norms_mm1(deg_parts, featsp, W1):
    return pl.pallas_call(
        _norms_mm1_body,
        grid=(GRID,),
        in_specs=[
            pl.BlockSpec((NW, 2, RB), lambda i: (0, 0, i)),
            pl.BlockSpec((RB, D), lambda i: (i, 0)),
            pl.BlockSpec((D, D), lambda i: (0, 0)),
        ],
        out_specs=[
            pl.BlockSpec((2, RB), lambda i: (0, i)),
            pl.BlockSpec((RB, D), lambda i: (i, 0)),
        ],
        out_shape=[
            jax.ShapeDtypeStruct((2, NP), jnp.float32),
            jax.ShapeDtypeStruct((NP, D), jnp.float32),
        ],
    )(deg_parts, featsp, W1)


def _layer_mm2_body(p_ref, n_ref, b_ref, w_ref, h_ref, y_ref):
    i = pl.program_id(0)
    agg = p_ref[0] + p_ref[1]
    nrm = n_ref[...]
    h = jnp.maximum(agg * nrm[1][:, None] + b_ref[...][None, :], 0.0)
    h = h * _rowmask(i)
    h_ref[...] = h
    y = jnp.dot(h, w_ref[...], preferred_element_type=jnp.float32)
    y_ref[...] = y * nrm[0][:, None]


def _layer_mm2(agg_parts, norms, b1, W2):
    return pl.pallas_call(
        _layer_mm2_body,
        grid=(GRID,),
        in_specs=[
            pl.BlockSpec((2, RB, D), lambda i: (0, i, 0)),
            pl.BlockSpec((2, RB), lambda i: (0, i)),
            pl.BlockSpec((D,), lambda i: (0,)),
            pl.BlockSpec((D, D), lambda i: (0, 0)),
        ],
        out_specs=[
            pl.BlockSpec((RB, D), lambda i: (i, 0)),
            pl.BlockSpec((RB, D), lambda i: (i, 0)),
        ],
        out_shape=[
            jax.ShapeDtypeStruct((NP, D), jnp.float32),
            jax.ShapeDtypeStruct((NP, D), jnp.float32),
        ],
    )(agg_parts, norms, b1, W2)


def _layer2_body(p_ref, n_ref, b_ref, h_ref):
    i = pl.program_id(0)
    agg = p_ref[0] + p_ref[1]
    nrm = n_ref[...]
    h = jnp.maximum(agg * nrm[1][:, None] + b_ref[...][None, :], 0.0)
    h_ref[...] = h * _rowmask(i)


def _layer2(agg_parts, norms, b2):
    return pl.pallas_call(
        _layer2_body,
        grid=(GRID,),
        in_specs=[
            pl.BlockSpec((2, RB, D), lambda i: (0, i, 0)),
            pl.BlockSpec((2, RB), lambda i: (0, i)),
            pl.BlockSpec((D,), lambda i: (0,)),
        ],
        out_specs=pl.BlockSpec((RB, D), lambda i: (i, 0)),
        out_shape=jax.ShapeDtypeStruct((NP, D), jnp.float32),
    )(agg_parts, norms, b2)


def _final_body(p1_ref, p2_ref, w_ref, b_ref, o_ref):
    a1 = p1_ref[0] + p1_ref[1]
    a2 = p2_ref[0] + p2_ref[1]
    w = w_ref[...]
    o = jnp.dot(a1, w[:D], preferred_element_type=jnp.float32)
    o += jnp.dot(a2, w[D:], preferred_element_type=jnp.float32)
    o_ref[...] = o + b_ref[...][None, :]


def _final(j1, j2, Wout, bout):
    return pl.pallas_call(
        _final_body,
        grid=(GRID,),
        in_specs=[
            pl.BlockSpec((2, RB, D), lambda i: (0, i, 0)),
            pl.BlockSpec((2, RB, D), lambda i: (0, i, 0)),
            pl.BlockSpec((2 * D, D), lambda i: (0, 0)),
            pl.BlockSpec((D,), lambda i: (0,)),
        ],
        out_specs=pl.BlockSpec((RB, D), lambda i: (i, 0)),
        out_shape=jax.ShapeDtypeStruct((N, D), jnp.float32),
    )(j1, j2, Wout, bout)


def kernel(feats, edge_index, W1, b1, W2, b2, Wout, bout):
    featsp = jnp.pad(feats, ((0, NP - N), (0, 0)))
    srcp = jnp.pad(edge_index[0], (0, EP - E), constant_values=N)
    dstp = jnp.pad(edge_index[1], (0, EP - E), constant_values=N)
    srcp3 = srcp.reshape(NW, EW_DEG)
    dstp3 = dstp.reshape(NW, EW_DEG)

    def _pack(a):
        a3 = a.reshape(NCHUNKS_TOT, 2, PKW)
        return a3[:, 0, :] | (a3[:, 1, :] << 16)

    pk = jnp.concatenate([_pack(srcp), _pack(dstp)], axis=1)
    pk = jnp.pad(pk, ((0, PKROWS - NCHUNKS_TOT), (0, 0)))

    deg_parts = _deg_kernel(srcp3, dstp3)
    norms, y1 = _norms_mm1(deg_parts, featsp, W1)
    agg1 = _agg_kernel(y1, pk)
    h1, y2 = _layer_mm2(agg1, norms, b1, W2)
    agg2 = _agg_kernel(y2, pk)
    h2 = _layer2(agg2, norms, b2)
    j1 = _agg_kernel(h1, pk)
    j2 = _agg_kernel(h2, pk)
    return _final(j1, j2, Wout, bout)
